# Initial kernel scaffold; baseline (speedup 1.0000x reference)
import jax
import jax.numpy as jnp
from jax.experimental import pallas as pl


def kernel(x, edge_index, hop_depths, last_hop_preagg, W1, b1, W2, b2, Wc, bc):
    raise NotImplementedError("write your pallas kernel here")


# speedup vs baseline: 12.8122x; 12.8122x over previous
import functools

import jax
import jax.numpy as jnp
from jax import lax
from jax.experimental import pallas as pl
from jax.experimental.pallas import tpu as pltpu
from jax.experimental.pallas import tpu_sc as plsc

_N = 10000
_E = 320000
_D = 128
_H = 128
_C = 64

_NC = 2
_NS = 16
_NW = _NC * _NS
_L = 16

_N_PAD = 10240
_TR = _N_PAD // _NS
_CHUNK = 128
_NCH = 79
_EPT = _NCH * _CHUNK
_E_PAD = _NW * _EPT
_DUMMY_DST = _N

_BR = 2048
_NB = _N_PAD // _BR

def _hist_body(dst_hbm, out_hbm, dstv, ones_v, zb_v, acc_sh):
    cid = lax.axis_index("c")
    sid = lax.axis_index("s")
    wid = cid * _NS + sid

    pltpu.sync_copy(dst_hbm.at[wid], dstv)

    def fill1(i, _):
        ones_v[pl.ds(i * _L, _L)] = jnp.ones((_L,), jnp.float32)
        return 0
    lax.fori_loop(0, _CHUNK // _L, fill1, 0)

    def fill0(i, _):
        zb_v[pl.ds(i * _L, _L)] = jnp.zeros((_L,), jnp.float32)
        return 0
    lax.fori_loop(0, _TR // _L, fill0, 0)

    pltpu.sync_copy(zb_v, acc_sh.at[pl.ds(sid * _TR, _TR)])
    plsc.subcore_barrier()

    def step(j, _):
        pltpu.sync_copy(ones_v, acc_sh.at[dstv.at[j]], add=True)
        return 0
    lax.fori_loop(0, _NCH, step, 0)

    plsc.subcore_barrier()
    pltpu.sync_copy(acc_sh.at[pl.ds(sid * _TR, _TR)],
                    out_hbm.at[cid, pl.ds(sid * _TR, _TR)])


@functools.cache
def _hist_kernel():
    return pl.kernel(
        _hist_body,
        out_type=jax.ShapeDtypeStruct((_NC, _N_PAD), jnp.float32),
        mesh=plsc.VectorSubcoreMesh(core_axis_name="c", subcore_axis_name="s"),
        scratch_types=[
            pltpu.VMEM((_NCH, _CHUNK), jnp.int32),
            pltpu.VMEM((_CHUNK,), jnp.float32),
            pltpu.VMEM((_TR,), jnp.float32),
            pltpu.VMEM_SHARED((_N_PAD,), jnp.float32),
        ],
    )


def _hist(dst3):
    return _hist_kernel()(dst3)


def _agg_body(y_hbm, src_hbm, dst_hbm, zeros_hbm, out_hbm,
              srcv, dstv, rows, acc_sh, sem):
    cid = lax.axis_index("c")
    sid = lax.axis_index("s")
    wid = cid * _NS + sid

    pltpu.sync_copy(src_hbm.at[wid], srcv)
    pltpu.sync_copy(dst_hbm.at[wid], dstv)

    pltpu.sync_copy(zeros_hbm, rows)
    for k in range(_TR // _CHUNK):
        pltpu.sync_copy(rows, acc_sh.at[pl.ds(sid * _TR + k * _CHUNK, _CHUNK)])
    plsc.subcore_barrier()

    def step(j, _):
        pltpu.async_copy(y_hbm.at[srcv.at[j]], rows, sem).wait()
        pltpu.sync_copy(rows, acc_sh.at[dstv.at[j]], add=True)
        return 0
    lax.fori_loop(0, _NCH, step, 0)

    plsc.subcore_barrier()
    for k in range(_TR // _CHUNK):
        sl = pl.ds(sid * _TR + k * _CHUNK, _CHUNK)
        pltpu.sync_copy(acc_sh.at[sl], rows)
        pltpu.sync_copy(rows, out_hbm.at[cid, sl])


@functools.cache
def _agg_kernel():
    return pl.kernel(
        _agg_body,
        out_type=jax.ShapeDtypeStruct((_NC, _N_PAD, _H), jnp.float32),
        mesh=plsc.VectorSubcoreMesh(core_axis_name="c", subcore_axis_name="s"),
        scratch_types=[
            pltpu.VMEM((_NCH, _CHUNK), jnp.int32),
            pltpu.VMEM((_NCH, _CHUNK), jnp.int32),
            pltpu.VMEM((_CHUNK, _H), jnp.float32),
            pltpu.VMEM_SHARED((_N_PAD, _H), jnp.float32),
            pltpu.SemaphoreType.DMA,
        ],
    )


def _agg(y, src3, dst3, zeros_tile):
    return _agg_kernel()(y, src3, dst3, zeros_tile)


def _k1_body(hop2d, hop_col, x_ref, pre_ref, w1_ref, d0_ref, d1_ref,
             y1_ref, premm_ref, dinv_ref):
    md = jnp.max(hop2d[...])
    hopb = hop_col[...]
    deepest = jnp.logical_and(md > 0, hopb == md)
    xb = jnp.where(deepest, 0.0, x_ref[...])
    dinv = lax.rsqrt(d0_ref[...] + d1_ref[...] + 1.0)
    y1_ref[...] = jnp.dot(xb, w1_ref[...],
                          preferred_element_type=jnp.float32) * dinv
    premm_ref[...] = jnp.dot(pre_ref[...], w1_ref[...],
                             preferred_element_type=jnp.float32)
    dinv_ref[...] = dinv


def _k3_body(hop2d, hop_col, a0_ref, a1_ref, y1_ref, premm_ref, dinv_ref,
             w2_ref, b1_ref, y2_ref):
    md = jnp.max(hop2d[...])
    frontier = jnp.logical_and(md > 0, hop_col[...] == md - 1)
    dinv = dinv_ref[...]
    agg = a0_ref[...] + a1_ref[...] + y1_ref[...]
    h1 = jnp.where(frontier, premm_ref[...], dinv * agg) + b1_ref[...]
    h1 = jnp.maximum(h1, 0.0)
    y2_ref[...] = jnp.dot(h1, w2_ref[...],
                          preferred_element_type=jnp.float32) * dinv


def _k5_body(p0_ref, p1_ref, y2_ref, dinv_ref, b2_ref, wc_ref, bc_ref,
             out_ref):
    agg = p0_ref[...] + p1_ref[...] + y2_ref[...]
    h2 = jnp.maximum(dinv_ref[...] * agg + b2_ref[...], 0.0)
    out_ref[...] = jnp.dot(h2, wc_ref[...],
                           preferred_element_type=jnp.float32) + bc_ref[...]


def _row_spec(w):
    return pl.BlockSpec((_BR, w), lambda i: (i, 0))


def _full_spec(h, w):
    return pl.BlockSpec((h, w), lambda i: (0, 0))


def kernel(x, edge_index, hop_depths, last_hop_preagg, W1, b1, W2, b2, Wc, bc):
    f32 = jnp.float32
    pad_n = _N_PAD - _N
    pad_e = _E_PAD - _E

    src3 = jnp.pad(edge_index[0], (0, pad_e)).reshape(_NW, _NCH, _CHUNK)
    dst3 = jnp.pad(edge_index[1], (0, pad_e),
                   constant_values=_DUMMY_DST).reshape(_NW, _NCH, _CHUNK)
    x_p = jnp.pad(x, ((0, pad_n), (0, 0)))
    pre_p = jnp.pad(last_hop_preagg, ((0, pad_n), (0, 0)))
    hop_p = jnp.pad(hop_depths, (0, pad_n))
    hop2d = hop_p.reshape(_N_PAD // _D, _D)
    hop_col = hop_p.reshape(_N_PAD, 1)
    zeros_tile = jnp.zeros((_CHUNK, _H), f32)
    b1r = b1.reshape(1, _H)
    b2r = b2.reshape(1, _H)
    bcr = bc.reshape(1, _C)

    deg_parts = _hist(dst3)
    d0 = deg_parts[0].reshape(_N_PAD, 1)
    d1 = deg_parts[1].reshape(_N_PAD, 1)

    y1, premm, dinv = pl.pallas_call(
        _k1_body,
        grid=(_NB,),
        in_specs=[_full_spec(_N_PAD // _D, _D), _row_spec(1), _row_spec(_D),
                  _row_spec(_D), _full_spec(_D, _H), _row_spec(1),
                  _row_spec(1)],
        out_specs=[_row_spec(_H), _row_spec(_H), _row_spec(1)],
        out_shape=[jax.ShapeDtypeStruct((_N_PAD, _H), f32),
                   jax.ShapeDtypeStruct((_N_PAD, _H), f32),
                   jax.ShapeDtypeStruct((_N_PAD, 1), f32)],
    )(hop2d, hop_col, x_p, pre_p, W1, d0, d1)

    parts1 = _agg(y1, src3, dst3, zeros_tile)

    y2 = pl.pallas_call(
        _k3_body,
        grid=(_NB,),
        in_specs=[_full_spec(_N_PAD // _D, _D), _row_spec(1), _row_spec(_H),
                  _row_spec(_H), _row_spec(_H), _row_spec(_H), _row_spec(1),
                  _full_spec(_H, _H), _full_spec(1, _H)],
        out_specs=[_row_spec(_H)],
        out_shape=[jax.ShapeDtypeStruct((_N_PAD, _H), f32)],
    )(hop2d, hop_col, parts1[0], parts1[1], y1, premm, dinv, W2, b1r)[0]

    parts2 = _agg(y2, src3, dst3, zeros_tile)

    out = pl.pallas_call(
        _k5_body,
        grid=(_NB,),
        in_specs=[_row_spec(_H), _row_spec(_H), _row_spec(_H), _row_spec(1),
                  _full_spec(1, _H), _full_spec(_H, _C), _full_spec(1, _C)],
        out_specs=[_row_spec(_C)],
        out_shape=[jax.ShapeDtypeStruct((_N_PAD, _C), f32)],
    )(parts2[0], parts2[1], y2, dinv, b2r, Wc, bcr)[0]

    return out[:_N]
